# baseline (speedup 1.0000x reference)
import functools
import jax
import jax.numpy as jnp
from jax import lax
from jax.experimental import pallas as pl
from jax.experimental.pallas import tpu as pltpu
from jax.experimental.pallas import tpu_sc as plsc

N = 10000
E = 160000
D_IN = 256
D_HID = 512
D_OUT = 40

NC, NS = 2, 16
NW = NC * NS
B = 128
NB = -(-E // (NW * B))
NB0 = 20
NB1 = 2 * NB - NB0
NBMAX = max(NB0, NB1)
E_PAD = NB * B * NW
N_ACC = 10240
RPT = N_ACC // NS
DEG_D = 16
ZROWS = 128

_f32 = jnp.float32


def _make_sc_agg(num_chunks, D, with_deg):
  mesh = plsc.VectorSubcoreMesh(core_axis_name="c", subcore_axis_name="s",
                                num_cores=NC, num_subcores=NS)
  out_type = [jax.ShapeDtypeStruct((NC, N_ACC, D), _f32)
              for _ in range(num_chunks)]
  if with_deg:
    out_type.append(jax.ShapeDtypeStruct((NC, N_ACC, DEG_D), _f32))

  scratch = [
      pltpu.VMEM((NBMAX, B), jnp.int32),
      pltpu.VMEM((NBMAX, B), jnp.int32),
      pltpu.VMEM((B, D), _f32),
      pltpu.VMEM((B, D), _f32),
      pltpu.VMEM((ZROWS, D), _f32),
      pltpu.VMEM_SHARED((N_ACC, D), _f32),
      pltpu.SemaphoreType.DMA,
      pltpu.SemaphoreType.DMA,
      pltpu.SemaphoreType.DMA,
      pltpu.SemaphoreType.DMA,
  ]
  if with_deg:
    scratch += [
        pltpu.VMEM((B, DEG_D), _f32),
        pltpu.VMEM((ZROWS, DEG_D), _f32),
        pltpu.VMEM_SHARED((N_ACC, DEG_D), _f32),
    ]

  def body(*refs):
    tabs = refs[:num_chunks]
    srcb, dstb = refs[num_chunks], refs[num_chunks + 1]
    k = num_chunks + 2
    outs = refs[k:k + num_chunks]
    k += num_chunks
    if with_deg:
      deg_out = refs[k]
      k += 1
    sidx, didx, msga, msgb, zb = refs[k:k + 5]
    acc = refs[k + 5]
    gsa, gsb, ssa, ssb = refs[k + 6:k + 10]
    if with_deg:
      ones, zbd, accd = refs[k + 10:k + 13]

    cid = lax.axis_index("c")
    sid = lax.axis_index("s")
    wid = sid * NC + cid
    nb = jnp.where(cid == 0, NB0, NB1)

    pltpu.sync_copy(srcb.at[wid], sidx)
    pltpu.sync_copy(dstb.at[wid], didx)

    def fill(ref, rows, d, val):
      v = jnp.full((16,), val, _f32)
      def st(i, _):
        r = i // (d // 16)
        c = i % (d // 16)
        ref[r, pl.ds(c * 16, 16)] = v
        return 0
      lax.fori_loop(0, rows * (d // 16), st, 0)

    fill(zb, ZROWS, D, 0.0)
    if with_deg:
      fill(ones, B, DEG_D, 1.0)
      fill(zbd, ZROWS, DEG_D, 0.0)

    def clear(a, z):
      hs = [pltpu.async_copy(z, a.at[pl.ds(sid * RPT + j * ZROWS, ZROWS)],
                             gsa) for j in range(RPT // ZROWS)]
      for h in hs:
        h.wait()

    for c in range(num_chunks):
      clear(acc, zb)
      if with_deg and c == 0:
        clear(accd, zbd)
      plsc.subcore_barrier()

      tab = tabs[c]
      deg_here = with_deg and c == 0

      def scatter_wait(j, buf, sem):
        hs = [pltpu.async_copy(buf, acc.at[didx.at[j]], sem, add=True)]
        if deg_here:
          hs.append(pltpu.async_copy(ones, accd.at[didx.at[j]], sem,
                                     add=True))
        for h in hs:
          h.wait()

      def drain_gather(buf, sem):
        pltpu.make_async_copy(tab.at[sidx.at[0]], buf, sem).wait()

      pltpu.async_copy(tab.at[sidx.at[0]], msga, gsa)

      def step(t, _):
        j0 = 2 * t
        j1 = j0 + 1
        pltpu.async_copy(tab.at[sidx.at[j1]], msgb, gsb)
        drain_gather(msga, gsa)
        scatter_wait(j0, msga, ssa)
        pltpu.async_copy(tab.at[sidx.at[jnp.minimum(j0 + 2, nb - 1)]],
                         msga, gsa)
        drain_gather(msgb, gsb)
        scatter_wait(j1, msgb, ssb)
        return 0

      lax.fori_loop(0, nb // 2, step, 0)
      drain_gather(msga, gsa)
      plsc.subcore_barrier()

      pltpu.sync_copy(acc.at[pl.ds(sid * RPT, RPT)],
                      outs[c].at[cid].at[pl.ds(sid * RPT, RPT)])
      if with_deg and c == 0:
        pltpu.sync_copy(accd.at[pl.ds(sid * RPT, RPT)],
                        deg_out.at[cid].at[pl.ds(sid * RPT, RPT)])
      if c + 1 < num_chunks:
        plsc.subcore_barrier()

  return pl.kernel(body, out_type=tuple(out_type), mesh=mesh,
                   scratch_types=scratch,
                   compiler_params=pltpu.CompilerParams(
                       use_tc_tiling_on_sc=False))


CD = 64
_sc_agg_l1 = _make_sc_agg(D_IN // CD, CD, True)
_sc_agg_l2 = _make_sc_agg(D_HID // CD, CD, False)
_sc_agg_l3 = _make_sc_agg(1, 48, False)

R = 400
GRID = N // R


def _inv_deg(pd):
  return 1.0 / jnp.maximum(pd[0, :, 0:1] + pd[1, :, 0:1], 1.0)


def _tc1_body(*refs):
  nc_in = D_IN // CD
  nc_out = D_HID // CD
  ps = refs[:nc_in]
  pd, w1, b1 = refs[nc_in:nc_in + 3]
  outs = refs[nc_in + 3:]
  inv = _inv_deg(pd[...])
  w = w1[...]
  h = b1[...]
  for c in range(nc_in):
    a = (ps[c][0] + ps[c][1]) * inv
    h += jnp.dot(a, w[c * CD:(c + 1) * CD], preferred_element_type=_f32)
  h = jnp.maximum(h, 0.0)
  for c in range(nc_out):
    outs[c][...] = h[:, c * CD:(c + 1) * CD]


def _tc2_body(*refs):
  nc_in = D_HID // CD
  qs = refs[:nc_in]
  pd, w2, b2, w3, z = refs[nc_in:]
  inv = _inv_deg(pd[...])
  w = w2[...]
  h = b2[...]
  for c in range(nc_in):
    a = (qs[c][0] + qs[c][1]) * inv
    h += jnp.dot(a, w[c * CD:(c + 1) * CD], preferred_element_type=_f32)
  h = jnp.maximum(h, 0.0)
  z[...] = jnp.dot(h, w3[...], preferred_element_type=_f32)


def _tc3_body(zp, pd, b3, o):
  inv = _inv_deg(pd[...])
  t = (zp[0] + zp[1]) * inv + b3[...]
  col = lax.broadcasted_iota(jnp.int32, t.shape, 1)
  valid = col < D_OUT
  tm = jnp.where(valid, t, -1e30)
  m = jnp.max(tm, axis=1, keepdims=True)
  e = jnp.where(valid, jnp.exp(t - m), 0.0)
  lse = jnp.log(jnp.sum(e, axis=1, keepdims=True))
  o[...] = t - m - lse


def _part_spec(d):
  return pl.BlockSpec((NC, R, d), lambda i: (0, i, 0))


def _full_spec(shape):
  return pl.BlockSpec(shape, lambda i: tuple(0 for _ in shape))


def _row_spec(d):
  return pl.BlockSpec((R, d), lambda i: (i, 0))


_tc1 = pl.pallas_call(
    _tc1_body,
    grid=(GRID,),
    in_specs=[_part_spec(CD)] * (D_IN // CD) + [_part_spec(DEG_D),
              _full_spec((D_IN, D_HID)), _full_spec((1, D_HID))],
    out_specs=[_row_spec(CD)] * (D_HID // CD),
    out_shape=[jax.ShapeDtypeStruct((N, CD), _f32)] * (D_HID // CD),
)

_tc2 = pl.pallas_call(
    _tc2_body,
    grid=(GRID,),
    in_specs=[_part_spec(CD)] * (D_HID // CD) + [_part_spec(DEG_D),
              _full_spec((D_HID, D_HID)), _full_spec((1, D_HID)),
              _full_spec((D_HID, 48))],
    out_specs=_row_spec(48),
    out_shape=jax.ShapeDtypeStruct((N, 48), _f32),
)

_tc3 = pl.pallas_call(
    _tc3_body,
    grid=(GRID,),
    in_specs=[_part_spec(48), _part_spec(DEG_D), _full_spec((1, 48))],
    out_specs=_row_spec(48),
    out_shape=jax.ShapeDtypeStruct((N, 48), _f32),
)


@jax.jit
def kernel(input_features, edge_index, W1, b1, W2, b2, W3, b3):
  x = input_features
  src = edge_index[0].astype(jnp.int32)
  dst = edge_index[1].astype(jnp.int32)

  def layout(v, fill):
    vp = jnp.concatenate([v, jnp.full((E_PAD - E,), fill, jnp.int32)])
    rows, off = [], 0
    for w in range(NW):
      n = (NB0 if w % NC == 0 else NB1) * B
      row = vp[off:off + n]
      off += n
      if n < NBMAX * B:
        row = jnp.concatenate([row, jnp.full((NBMAX * B - n,), fill,
                                             jnp.int32)])
      rows.append(row.reshape(NBMAX, B))
    return jnp.stack(rows)

  srcp = layout(src, 0)
  dstp = layout(dst, N_ACC - 1)

  xc = [x[:, c * CD:(c + 1) * CD] for c in range(D_IN // CD)]
  *ps, pdeg = _sc_agg_l1(*xc, srcp, dstp)
  hs = _tc1(*ps, pdeg, W1, b1.reshape(1, -1))
  qs = _sc_agg_l2(*hs, srcp, dstp)
  z = _tc2(*qs, pdeg, W2, b2.reshape(1, -1),
           jnp.pad(W3, ((0, 0), (0, 48 - D_OUT))))
  zp, = _sc_agg_l3(z, srcp, dstp)
  o = _tc3(zp, pdeg, jnp.pad(b3, (0, 48 - D_OUT)).reshape(1, -1))
  return o[:, :D_OUT]


# speedup vs baseline: 1.3467x; 1.3467x over previous
import functools
import jax
import jax.numpy as jnp
from jax import lax
from jax.experimental import pallas as pl
from jax.experimental.pallas import tpu as pltpu
from jax.experimental.pallas import tpu_sc as plsc

N = 10000
E = 160000
D_IN = 256
D_HID = 512
D_OUT = 40

NC, NS = 2, 16
NW = NC * NS
B = 128
NB = -(-E // (NW * B))
NB0 = 60
NB1 = 2 * NB - NB0
NBMAX = max(NB0, NB1)
E_PAD = NB * B * NW
N_ACC = 10240
RPT = N_ACC // NS
DEG_D = 16
ZROWS = 128

_f32 = jnp.float32


def _make_sc_agg(num_chunks, D, with_deg):
  mesh = plsc.VectorSubcoreMesh(core_axis_name="c", subcore_axis_name="s",
                                num_cores=NC, num_subcores=NS)
  out_type = [jax.ShapeDtypeStruct((NC, N_ACC, D), _f32)
              for _ in range(num_chunks)]
  if with_deg:
    out_type.append(jax.ShapeDtypeStruct((NC, N_ACC, DEG_D), _f32))

  scratch = [
      pltpu.VMEM((NBMAX, B), jnp.int32),
      pltpu.VMEM((NBMAX, B), jnp.int32),
      pltpu.VMEM((B, D), _f32),
      pltpu.VMEM((B, D), _f32),
      pltpu.VMEM((ZROWS, D), _f32),
      pltpu.VMEM_SHARED((N_ACC, D), _f32),
      pltpu.SemaphoreType.DMA,
      pltpu.SemaphoreType.DMA,
      pltpu.SemaphoreType.DMA,
      pltpu.SemaphoreType.DMA,
  ]
  if with_deg:
    scratch += [
        pltpu.VMEM((B, DEG_D), _f32),
        pltpu.VMEM((ZROWS, DEG_D), _f32),
        pltpu.VMEM_SHARED((N_ACC, DEG_D), _f32),
    ]

  def body(*refs):
    tabs = refs[:num_chunks]
    srcb, dstb = refs[num_chunks], refs[num_chunks + 1]
    k = num_chunks + 2
    outs = refs[k:k + num_chunks]
    k += num_chunks
    if with_deg:
      deg_out = refs[k]
      k += 1
    sidx, didx, msga, msgb, zb = refs[k:k + 5]
    acc = refs[k + 5]
    gsa, gsb, ssa, ssb = refs[k + 6:k + 10]
    if with_deg:
      ones, zbd, accd = refs[k + 10:k + 13]

    cid = lax.axis_index("c")
    sid = lax.axis_index("s")
    wid = sid * NC + cid
    nb = jnp.where(cid == 0, NB0, NB1)

    pltpu.sync_copy(srcb.at[wid], sidx)
    pltpu.sync_copy(dstb.at[wid], didx)

    def fill(ref, rows, d, val):
      v = jnp.full((16,), val, _f32)
      def st(i, _):
        r = i // (d // 16)
        c = i % (d // 16)
        ref[r, pl.ds(c * 16, 16)] = v
        return 0
      lax.fori_loop(0, rows * (d // 16), st, 0)

    fill(zb, ZROWS, D, 0.0)
    if with_deg:
      fill(ones, B, DEG_D, 1.0)
      fill(zbd, ZROWS, DEG_D, 0.0)

    def clear(a, z):
      hs = [pltpu.async_copy(z, a.at[pl.ds(sid * RPT + j * ZROWS, ZROWS)],
                             gsa) for j in range(RPT // ZROWS)]
      for h in hs:
        h.wait()

    for c in range(num_chunks):
      clear(acc, zb)
      if with_deg and c == 0:
        clear(accd, zbd)
      plsc.subcore_barrier()

      tab = tabs[c]
      deg_here = with_deg and c == 0

      def scatter_wait(j, buf, sem):
        hs = [pltpu.async_copy(buf, acc.at[didx.at[j]], sem, add=True)]
        if deg_here:
          hs.append(pltpu.async_copy(ones, accd.at[didx.at[j]], sem,
                                     add=True))
        for h in hs:
          h.wait()

      def drain_gather(buf, sem):
        pltpu.make_async_copy(tab.at[sidx.at[0]], buf, sem).wait()

      pltpu.async_copy(tab.at[sidx.at[0]], msga, gsa)

      def step(t, _):
        j0 = 2 * t
        j1 = j0 + 1
        pltpu.async_copy(tab.at[sidx.at[j1]], msgb, gsb)
        drain_gather(msga, gsa)
        scatter_wait(j0, msga, ssa)
        pltpu.async_copy(tab.at[sidx.at[jnp.minimum(j0 + 2, nb - 1)]],
                         msga, gsa)
        drain_gather(msgb, gsb)
        scatter_wait(j1, msgb, ssb)
        return 0

      lax.fori_loop(0, nb // 2, step, 0)
      drain_gather(msga, gsa)
      plsc.subcore_barrier()

      pltpu.sync_copy(acc.at[pl.ds(sid * RPT, RPT)],
                      outs[c].at[cid].at[pl.ds(sid * RPT, RPT)])
      if with_deg and c == 0:
        pltpu.sync_copy(accd.at[pl.ds(sid * RPT, RPT)],
                        deg_out.at[cid].at[pl.ds(sid * RPT, RPT)])
      if c + 1 < num_chunks:
        plsc.subcore_barrier()

  return pl.kernel(body, out_type=tuple(out_type), mesh=mesh,
                   scratch_types=scratch,
                   compiler_params=pltpu.CompilerParams(
                       use_tc_tiling_on_sc=False))


CD = 64
_sc_agg_l1 = _make_sc_agg(D_IN // CD, CD, True)
_sc_agg_l2 = _make_sc_agg(D_HID // CD, CD, False)
_sc_agg_l3 = _make_sc_agg(1, 48, False)

R = 400
GRID = N // R


def _inv_deg(pd):
  return 1.0 / jnp.maximum(pd[0, :, 0:1] + pd[1, :, 0:1], 1.0)


def _tc1_body(*refs):
  nc_in = D_IN // CD
  nc_out = D_HID // CD
  ps = refs[:nc_in]
  pd, w1, b1 = refs[nc_in:nc_in + 3]
  outs = refs[nc_in + 3:]
  inv = _inv_deg(pd[...])
  w = w1[...]
  h = b1[...]
  for c in range(nc_in):
    a = (ps[c][0] + ps[c][1]) * inv
    h += jnp.dot(a, w[c * CD:(c + 1) * CD], preferred_element_type=_f32)
  h = jnp.maximum(h, 0.0)
  for c in range(nc_out):
    outs[c][...] = h[:, c * CD:(c + 1) * CD]


def _tc2_body(*refs):
  nc_in = D_HID // CD
  qs = refs[:nc_in]
  pd, w2, b2, w3, z = refs[nc_in:]
  inv = _inv_deg(pd[...])
  w = w2[...]
  h = b2[...]
  for c in range(nc_in):
    a = (qs[c][0] + qs[c][1]) * inv
    h += jnp.dot(a, w[c * CD:(c + 1) * CD], preferred_element_type=_f32)
  h = jnp.maximum(h, 0.0)
  z[...] = jnp.dot(h, w3[...], preferred_element_type=_f32)


def _tc3_body(zp, pd, b3, o):
  inv = _inv_deg(pd[...])
  t = (zp[0] + zp[1]) * inv + b3[...]
  col = lax.broadcasted_iota(jnp.int32, t.shape, 1)
  valid = col < D_OUT
  tm = jnp.where(valid, t, -1e30)
  m = jnp.max(tm, axis=1, keepdims=True)
  e = jnp.where(valid, jnp.exp(t - m), 0.0)
  lse = jnp.log(jnp.sum(e, axis=1, keepdims=True))
  o[...] = t - m - lse


def _part_spec(d):
  return pl.BlockSpec((NC, R, d), lambda i: (0, i, 0))


def _full_spec(shape):
  return pl.BlockSpec(shape, lambda i: tuple(0 for _ in shape))


def _row_spec(d):
  return pl.BlockSpec((R, d), lambda i: (i, 0))


_tc1 = pl.pallas_call(
    _tc1_body,
    grid=(GRID,),
    in_specs=[_part_spec(CD)] * (D_IN // CD) + [_part_spec(DEG_D),
              _full_spec((D_IN, D_HID)), _full_spec((1, D_HID))],
    out_specs=[_row_spec(CD)] * (D_HID // CD),
    out_shape=[jax.ShapeDtypeStruct((N, CD), _f32)] * (D_HID // CD),
)

_tc2 = pl.pallas_call(
    _tc2_body,
    grid=(GRID,),
    in_specs=[_part_spec(CD)] * (D_HID // CD) + [_part_spec(DEG_D),
              _full_spec((D_HID, D_HID)), _full_spec((1, D_HID)),
              _full_spec((D_HID, 48))],
    out_specs=_row_spec(48),
    out_shape=jax.ShapeDtypeStruct((N, 48), _f32),
)

_tc3 = pl.pallas_call(
    _tc3_body,
    grid=(GRID,),
    in_specs=[_part_spec(48), _part_spec(DEG_D), _full_spec((1, 48))],
    out_specs=_row_spec(48),
    out_shape=jax.ShapeDtypeStruct((N, 48), _f32),
)


@jax.jit
def kernel(input_features, edge_index, W1, b1, W2, b2, W3, b3):
  x = input_features
  src = edge_index[0].astype(jnp.int32)
  dst = edge_index[1].astype(jnp.int32)

  def layout(v, fill):
    vp = jnp.concatenate([v, jnp.full((E_PAD - E,), fill, jnp.int32)])
    rows, off = [], 0
    for w in range(NW):
      n = (NB0 if w % NC == 0 else NB1) * B
      row = vp[off:off + n]
      off += n
      if n < NBMAX * B:
        row = jnp.concatenate([row, jnp.full((NBMAX * B - n,), fill,
                                             jnp.int32)])
      rows.append(row.reshape(NBMAX, B))
    return jnp.stack(rows)

  srcp = layout(src, 0)
  dstp = layout(dst, N_ACC - 1)

  xc = [x[:, c * CD:(c + 1) * CD] for c in range(D_IN // CD)]
  *ps, pdeg = _sc_agg_l1(*xc, srcp, dstp)
  hs = _tc1(*ps, pdeg, W1, b1.reshape(1, -1))
  qs = _sc_agg_l2(*hs, srcp, dstp)
  z = _tc2(*qs, pdeg, W2, b2.reshape(1, -1),
           jnp.pad(W3, ((0, 0), (0, 48 - D_OUT))))
  zp, = _sc_agg_l3(z, srcp, dstp)
  o = _tc3(zp, pdeg, jnp.pad(b3, (0, 48 - D_OUT)).reshape(1, -1))
  return o[:, :D_OUT]


# speedup vs baseline: 2.1736x; 1.6140x over previous
import functools
import jax
import jax.numpy as jnp
from jax import lax
from jax.experimental import pallas as pl
from jax.experimental.pallas import tpu as pltpu
from jax.experimental.pallas import tpu_sc as plsc

N = 10000
E = 160000
D_IN = 256
D_HID = 512
D_OUT = 40

NC, NS = 2, 16
NW = NC * NS
B = 64
NB = -(-E // (NW * B))
NB0 = 118
NB1 = 2 * NB - NB0
NBMAX = max(NB0, NB1)
E_PAD = NB * B * NW
N_ACC = 10240
RPT = N_ACC // NS
DEG_D = 16
ZROWS = 32

_f32 = jnp.float32


def _make_sc_agg(num_chunks, D, with_deg):
  mesh = plsc.VectorSubcoreMesh(core_axis_name="c", subcore_axis_name="s",
                                num_cores=NC, num_subcores=NS)
  out_type = [jax.ShapeDtypeStruct((NC, N_ACC, D), _f32)
              for _ in range(num_chunks)]
  if with_deg:
    out_type.append(jax.ShapeDtypeStruct((NC, N_ACC, DEG_D), _f32))

  scratch = [
      pltpu.VMEM((NBMAX, B), jnp.int32),
      pltpu.VMEM((NBMAX, B), jnp.int32),
      pltpu.VMEM((B, D), _f32),
      pltpu.VMEM((B, D), _f32),
      pltpu.VMEM((ZROWS, D), _f32),
      pltpu.VMEM_SHARED((N_ACC, D), _f32),
      pltpu.SemaphoreType.DMA,
      pltpu.SemaphoreType.DMA,
      pltpu.SemaphoreType.DMA,
      pltpu.SemaphoreType.DMA,
  ]
  if with_deg:
    scratch += [
        pltpu.VMEM((B, DEG_D), _f32),
        pltpu.VMEM((ZROWS, DEG_D), _f32),
        pltpu.VMEM_SHARED((N_ACC, DEG_D), _f32),
    ]

  def body(*refs):
    tabs = refs[:num_chunks]
    srcb, dstb = refs[num_chunks], refs[num_chunks + 1]
    k = num_chunks + 2
    outs = refs[k:k + num_chunks]
    k += num_chunks
    if with_deg:
      deg_out = refs[k]
      k += 1
    sidx, didx, msga, msgb, zb = refs[k:k + 5]
    acc = refs[k + 5]
    gsa, gsb, ssa, ssb = refs[k + 6:k + 10]
    if with_deg:
      ones, zbd, accd = refs[k + 10:k + 13]

    cid = lax.axis_index("c")
    sid = lax.axis_index("s")
    wid = sid * NC + cid
    nb = jnp.where(cid == 0, NB0, NB1)

    pltpu.sync_copy(srcb.at[wid], sidx)
    pltpu.sync_copy(dstb.at[wid], didx)

    def fill(ref, rows, d, val):
      v = jnp.full((16,), val, _f32)
      def st(i, _):
        r = i // (d // 16)
        c = i % (d // 16)
        ref[r, pl.ds(c * 16, 16)] = v
        return 0
      lax.fori_loop(0, rows * (d // 16), st, 0)

    fill(zb, ZROWS, D, 0.0)
    if with_deg:
      fill(ones, B, DEG_D, 1.0)
      fill(zbd, ZROWS, DEG_D, 0.0)

    def clear(a, z):
      hs = [pltpu.async_copy(z, a.at[pl.ds(sid * RPT + j * ZROWS, ZROWS)],
                             gsa) for j in range(RPT // ZROWS)]
      for h in hs:
        h.wait()

    for c in range(num_chunks):
      clear(acc, zb)
      if with_deg and c == 0:
        clear(accd, zbd)
      plsc.subcore_barrier()

      tab = tabs[c]
      deg_here = with_deg and c == 0

      def scatter_wait(j, buf, sem):
        hs = [pltpu.async_copy(buf, acc.at[didx.at[j]], sem, add=True)]
        if deg_here:
          hs.append(pltpu.async_copy(ones, accd.at[didx.at[j]], sem,
                                     add=True))
        for h in hs:
          h.wait()

      def drain_gather(buf, sem):
        pltpu.make_async_copy(tab.at[sidx.at[0]], buf, sem).wait()

      pltpu.async_copy(tab.at[sidx.at[0]], msga, gsa)

      def step(t, _):
        j0 = 2 * t
        j1 = j0 + 1
        pltpu.async_copy(tab.at[sidx.at[j1]], msgb, gsb)
        drain_gather(msga, gsa)
        scatter_wait(j0, msga, ssa)
        pltpu.async_copy(tab.at[sidx.at[jnp.minimum(j0 + 2, nb - 1)]],
                         msga, gsa)
        drain_gather(msgb, gsb)
        scatter_wait(j1, msgb, ssb)
        return 0

      lax.fori_loop(0, nb // 2, step, 0)
      drain_gather(msga, gsa)
      plsc.subcore_barrier()

      pltpu.sync_copy(acc.at[pl.ds(sid * RPT, RPT)],
                      outs[c].at[cid].at[pl.ds(sid * RPT, RPT)])
      if with_deg and c == 0:
        pltpu.sync_copy(accd.at[pl.ds(sid * RPT, RPT)],
                        deg_out.at[cid].at[pl.ds(sid * RPT, RPT)])
      if c + 1 < num_chunks:
        plsc.subcore_barrier()

  return pl.kernel(body, out_type=tuple(out_type), mesh=mesh,
                   scratch_types=scratch,
                   compiler_params=pltpu.CompilerParams(
                       use_tc_tiling_on_sc=False))


CD = 128
_sc_agg_l1 = _make_sc_agg(D_IN // CD, CD, True)
_sc_agg_l2 = _make_sc_agg(D_HID // CD, CD, False)
_sc_agg_l3 = _make_sc_agg(1, 48, False)

R = 400
GRID = N // R


def _inv_deg(pd):
  return 1.0 / jnp.maximum(pd[0, :, 0:1] + pd[1, :, 0:1], 1.0)


def _tc1_body(*refs):
  nc_in = D_IN // CD
  nc_out = D_HID // CD
  ps = refs[:nc_in]
  pd, w1, b1 = refs[nc_in:nc_in + 3]
  outs = refs[nc_in + 3:]
  inv = _inv_deg(pd[...])
  w = w1[...]
  h = b1[...]
  for c in range(nc_in):
    a = (ps[c][0] + ps[c][1]) * inv
    h += jnp.dot(a, w[c * CD:(c + 1) * CD], preferred_element_type=_f32)
  h = jnp.maximum(h, 0.0)
  for c in range(nc_out):
    outs[c][...] = h[:, c * CD:(c + 1) * CD]


def _tc2_body(*refs):
  nc_in = D_HID // CD
  qs = refs[:nc_in]
  pd, w2, b2, w3, z = refs[nc_in:]
  inv = _inv_deg(pd[...])
  w = w2[...]
  h = b2[...]
  for c in range(nc_in):
    a = (qs[c][0] + qs[c][1]) * inv
    h += jnp.dot(a, w[c * CD:(c + 1) * CD], preferred_element_type=_f32)
  h = jnp.maximum(h, 0.0)
  z[...] = jnp.dot(h, w3[...], preferred_element_type=_f32)


def _tc3_body(zp, pd, b3, o):
  inv = _inv_deg(pd[...])
  t = (zp[0] + zp[1]) * inv + b3[...]
  col = lax.broadcasted_iota(jnp.int32, t.shape, 1)
  valid = col < D_OUT
  tm = jnp.where(valid, t, -1e30)
  m = jnp.max(tm, axis=1, keepdims=True)
  e = jnp.where(valid, jnp.exp(t - m), 0.0)
  lse = jnp.log(jnp.sum(e, axis=1, keepdims=True))
  o[...] = t - m - lse


def _part_spec(d):
  return pl.BlockSpec((NC, R, d), lambda i: (0, i, 0))


def _full_spec(shape):
  return pl.BlockSpec(shape, lambda i: tuple(0 for _ in shape))


def _row_spec(d):
  return pl.BlockSpec((R, d), lambda i: (i, 0))


_tc1 = pl.pallas_call(
    _tc1_body,
    grid=(GRID,),
    in_specs=[_part_spec(CD)] * (D_IN // CD) + [_part_spec(DEG_D),
              _full_spec((D_IN, D_HID)), _full_spec((1, D_HID))],
    out_specs=[_row_spec(CD)] * (D_HID // CD),
    out_shape=[jax.ShapeDtypeStruct((N, CD), _f32)] * (D_HID // CD),
)

_tc2 = pl.pallas_call(
    _tc2_body,
    grid=(GRID,),
    in_specs=[_part_spec(CD)] * (D_HID // CD) + [_part_spec(DEG_D),
              _full_spec((D_HID, D_HID)), _full_spec((1, D_HID)),
              _full_spec((D_HID, 48))],
    out_specs=_row_spec(48),
    out_shape=jax.ShapeDtypeStruct((N, 48), _f32),
)

_tc3 = pl.pallas_call(
    _tc3_body,
    grid=(GRID,),
    in_specs=[_part_spec(48), _part_spec(DEG_D), _full_spec((1, 48))],
    out_specs=_row_spec(48),
    out_shape=jax.ShapeDtypeStruct((N, 48), _f32),
)


@jax.jit
def kernel(input_features, edge_index, W1, b1, W2, b2, W3, b3):
  x = input_features
  src = edge_index[0].astype(jnp.int32)
  dst = edge_index[1].astype(jnp.int32)

  def layout(v, fill):
    vp = jnp.concatenate([v, jnp.full((E_PAD - E,), fill, jnp.int32)])
    rows, off = [], 0
    for w in range(NW):
      n = (NB0 if w % NC == 0 else NB1) * B
      row = vp[off:off + n]
      off += n
      if n < NBMAX * B:
        row = jnp.concatenate([row, jnp.full((NBMAX * B - n,), fill,
                                             jnp.int32)])
      rows.append(row.reshape(NBMAX, B))
    return jnp.stack(rows)

  srcp = layout(src, 0)
  dstp = layout(dst, N_ACC - 1)

  xc = [x[:, c * CD:(c + 1) * CD] for c in range(D_IN // CD)]
  *ps, pdeg = _sc_agg_l1(*xc, srcp, dstp)
  hs = _tc1(*ps, pdeg, W1, b1.reshape(1, -1))
  qs = _sc_agg_l2(*hs, srcp, dstp)
  z = _tc2(*qs, pdeg, W2, b2.reshape(1, -1),
           jnp.pad(W3, ((0, 0), (0, 48 - D_OUT))))
  zp, = _sc_agg_l3(z, srcp, dstp)
  o = _tc3(zp, pdeg, jnp.pad(b3, (0, 48 - D_OUT)).reshape(1, -1))
  return o[:, :D_OUT]
